# baseline (speedup 1.0000x reference)
import jax
import jax.numpy as jnp
from jax.experimental import pallas as pl
from jax.experimental.pallas import tpu as pltpu

_TAU_INV = 2.0
_NBUF = 4
_NSPLIT = 4
_NCHUNK = 4


def _start_row(u_hbm, buf, sems, row, slot, D):
    Ds = D // _NSPLIT
    for j in range(_NSPLIT):
        pltpu.make_async_copy(
            u_hbm.at[row, :, pl.ds(j * Ds, Ds)],
            buf.at[slot, :, pl.ds(j * Ds, Ds)],
            sems.at[slot, j],
        ).start(priority=j % 2)


def _wait_row(u_hbm, buf, sems, row, slot, D):
    Ds = D // _NSPLIT
    for j in range(_NSPLIT):
        pltpu.make_async_copy(
            u_hbm.at[row, :, pl.ds(j * Ds, Ds)],
            buf.at[slot, :, pl.ds(j * Ds, Ds)],
            sems.at[slot, j],
        ).wait()


def _body(l_ref, u_hbm, o_ref, buf, sems):
    b = pl.program_id(0)
    n = pl.num_programs(0)
    D = u_hbm.shape[2]
    slot = jax.lax.rem(b, _NBUF)

    @pl.when(b == 0)
    def _prologue():
        for j in range(_NBUF):
            _start_row(u_hbm, buf, sems, j, j, D)

    _wait_row(u_hbm, buf, sems, b, slot, D)

    Dc = D // _NCHUNK
    ars = []
    s = None
    for i in range(_NCHUNK):
        a = jnp.exp(l_ref[0, :, i * Dc:(i + 1) * Dc] * _TAU_INV)
        t = jnp.log(buf[slot, :, i * Dc:(i + 1) * Dc])
        ar = a / (t * t)
        ars.append(ar)
        p = jnp.sum(ar, axis=1, keepdims=True)
        s = p if s is None else s + p
    r = 1.0 / s
    for i, ar in enumerate(ars):
        o_ref[0, :, i * Dc:(i + 1) * Dc] = jnp.max(
            ar * r, axis=0, keepdims=True)

    b2 = b + _NBUF

    @pl.when(b2 < n)
    def _refill():
        _start_row(u_hbm, buf, sems, b2, jax.lax.rem(b2, _NBUF), D)


def kernel(logits, uniform):
    B, K, D = uniform.shape
    out = pl.pallas_call(
        _body,
        grid=(B,),
        in_specs=[
            pl.BlockSpec((1, 1, D), lambda b: (b, 0, 0)),
            pl.BlockSpec(memory_space=pltpu.HBM),
        ],
        out_specs=pl.BlockSpec((1, 1, D), lambda b: (b, 0, 0)),
        out_shape=jax.ShapeDtypeStruct((B, 1, D), jnp.float32),
        scratch_shapes=[
            pltpu.VMEM((_NBUF, K, D), jnp.float32),
            pltpu.SemaphoreType.DMA((_NBUF, _NSPLIT)),
        ],
        compiler_params=pltpu.CompilerParams(
            dimension_semantics=("arbitrary",)),
    )(logits.reshape(B, 1, D), uniform)
    return out.reshape(B, D)


# speedup vs baseline: 1.0104x; 1.0104x over previous
import jax
import jax.numpy as jnp
from jax.experimental import pallas as pl
from jax.experimental.pallas import tpu as pltpu

_TAU_INV = 2.0
_NBUF = 4
_NSPLIT = 4
_NOUT = 4
_NCHUNK = 4


def _u_copy(u_hbm, buf, sems, row, slot, j, D):
    Ds = D // _NSPLIT
    return pltpu.make_async_copy(
        u_hbm.at[row, :, pl.ds(j * Ds, Ds)],
        buf.at[slot, :, pl.ds(j * Ds, Ds)],
        sems.at[slot, j],
    )


def _body(l_hbm, u_hbm, o_hbm, buf, lbuf, obuf, sems, lsems, osems):
    B, K, D = u_hbm.shape

    for r in range(_NBUF):
        for j in range(_NSPLIT):
            _u_copy(u_hbm, buf, sems, r, r, j, D).start(priority=j % 2)
        pltpu.make_async_copy(l_hbm.at[r], lbuf.at[r], lsems.at[r]).start()

    def step(b, carry):
        slot = jax.lax.rem(b, _NBUF)
        for j in range(_NSPLIT):
            _u_copy(u_hbm, buf, sems, b, slot, j, D).wait()
        pltpu.make_async_copy(l_hbm.at[b], lbuf.at[slot], lsems.at[slot]).wait()

        a = jnp.exp(lbuf[slot] * _TAU_INV)
        Dc = D // _NCHUNK
        ars = []
        s = None
        for i in range(_NCHUNK):
            t = jnp.log(buf[slot, :, i * Dc:(i + 1) * Dc])
            ar = a[:, i * Dc:(i + 1) * Dc] / (t * t)
            ars.append(ar)
            p = jnp.sum(ar, axis=1, keepdims=True)
            s = p if s is None else s + p
        r_ = 1.0 / s
        m = jnp.concatenate(
            [jnp.max(ar * r_, axis=0, keepdims=True) for ar in ars],
            axis=1)

        oslot = jax.lax.rem(b, _NOUT)

        @pl.when(b >= _NOUT)
        def _drain_prev():
            pltpu.make_async_copy(
                obuf.at[oslot], o_hbm.at[b - _NOUT], osems.at[oslot]).wait()

        obuf[oslot] = m
        pltpu.make_async_copy(
            obuf.at[oslot], o_hbm.at[b], osems.at[oslot]).start()

        b2 = b + _NBUF

        @pl.when(b2 < B)
        def _refill():
            slot2 = jax.lax.rem(b2, _NBUF)
            for j in range(_NSPLIT):
                _u_copy(u_hbm, buf, sems, b2, slot2, j, D).start(
                    priority=j % 2)
            pltpu.make_async_copy(
                l_hbm.at[b2], lbuf.at[slot2], lsems.at[slot2]).start()

        return carry

    jax.lax.fori_loop(0, B, step, 0)

    for t in range(_NOUT):
        row = B - _NOUT + t
        pltpu.make_async_copy(
            obuf.at[row % _NOUT], o_hbm.at[row],
            osems.at[row % _NOUT]).wait()


def kernel(logits, uniform):
    B, K, D = uniform.shape
    out = pl.pallas_call(
        _body,
        in_specs=[
            pl.BlockSpec(memory_space=pltpu.HBM),
            pl.BlockSpec(memory_space=pltpu.HBM),
        ],
        out_specs=pl.BlockSpec(memory_space=pltpu.HBM),
        out_shape=jax.ShapeDtypeStruct((B, 1, D), jnp.float32),
        scratch_shapes=[
            pltpu.VMEM((_NBUF, K, D), jnp.float32),
            pltpu.VMEM((_NBUF, 1, D), jnp.float32),
            pltpu.VMEM((_NOUT, 1, D), jnp.float32),
            pltpu.SemaphoreType.DMA((_NBUF, _NSPLIT)),
            pltpu.SemaphoreType.DMA((_NBUF,)),
            pltpu.SemaphoreType.DMA((_NOUT,)),
        ],
    )(logits.reshape(B, 1, D), uniform)
    return out.reshape(B, D)


# speedup vs baseline: 1.0115x; 1.0011x over previous
import jax
import jax.numpy as jnp
from jax.experimental import pallas as pl
from jax.experimental.pallas import tpu as pltpu

_TAU_INV = 2.0
_NBUF = 4
_NSPLIT = 2
_NOUT = 4
_NCHUNK = 4


def _u_copy(u_hbm, buf, sems, row, slot, j, D):
    Ds = D // _NSPLIT
    return pltpu.make_async_copy(
        u_hbm.at[row, :, pl.ds(j * Ds, Ds)],
        buf.at[slot, :, pl.ds(j * Ds, Ds)],
        sems.at[slot, j],
    )


def _body(l_hbm, u_hbm, o_hbm, buf, lbuf, obuf, sems, lsems, osems):
    B, K, D = u_hbm.shape

    for r in range(_NBUF):
        for j in range(_NSPLIT):
            _u_copy(u_hbm, buf, sems, r, r, j, D).start(priority=j % 2)
        pltpu.make_async_copy(l_hbm.at[r], lbuf.at[r], lsems.at[r]).start()

    def step(b, carry):
        slot = jax.lax.rem(b, _NBUF)
        for j in range(_NSPLIT):
            _u_copy(u_hbm, buf, sems, b, slot, j, D).wait()
        pltpu.make_async_copy(l_hbm.at[b], lbuf.at[slot], lsems.at[slot]).wait()

        a = jnp.exp(lbuf[slot] * _TAU_INV)
        Dc = D // _NCHUNK
        ars = []
        s = None
        for i in range(_NCHUNK):
            t = jnp.log(buf[slot, :, i * Dc:(i + 1) * Dc])
            ar = a[:, i * Dc:(i + 1) * Dc] / (t * t)
            ars.append(ar)
            p = jnp.sum(ar, axis=1, keepdims=True)
            s = p if s is None else s + p
        r_ = 1.0 / s
        m = jnp.concatenate(
            [jnp.max(ar * r_, axis=0, keepdims=True) for ar in ars],
            axis=1)

        oslot = jax.lax.rem(b, _NOUT)

        @pl.when(b >= _NOUT)
        def _drain_prev():
            pltpu.make_async_copy(
                obuf.at[oslot], o_hbm.at[b - _NOUT], osems.at[oslot]).wait()

        obuf[oslot] = m
        pltpu.make_async_copy(
            obuf.at[oslot], o_hbm.at[b], osems.at[oslot]).start()

        b2 = b + _NBUF

        @pl.when(b2 < B)
        def _refill():
            slot2 = jax.lax.rem(b2, _NBUF)
            for j in range(_NSPLIT):
                _u_copy(u_hbm, buf, sems, b2, slot2, j, D).start(
                    priority=j % 2)
            pltpu.make_async_copy(
                l_hbm.at[b2], lbuf.at[slot2], lsems.at[slot2]).start()

        return carry

    jax.lax.fori_loop(0, B, step, 0)

    for t in range(_NOUT):
        row = B - _NOUT + t
        pltpu.make_async_copy(
            obuf.at[row % _NOUT], o_hbm.at[row],
            osems.at[row % _NOUT]).wait()


def kernel(logits, uniform):
    B, K, D = uniform.shape
    out = pl.pallas_call(
        _body,
        in_specs=[
            pl.BlockSpec(memory_space=pltpu.HBM),
            pl.BlockSpec(memory_space=pltpu.HBM),
        ],
        out_specs=pl.BlockSpec(memory_space=pltpu.HBM),
        out_shape=jax.ShapeDtypeStruct((B, 1, D), jnp.float32),
        scratch_shapes=[
            pltpu.VMEM((_NBUF, K, D), jnp.float32),
            pltpu.VMEM((_NBUF, 1, D), jnp.float32),
            pltpu.VMEM((_NOUT, 1, D), jnp.float32),
            pltpu.SemaphoreType.DMA((_NBUF, _NSPLIT)),
            pltpu.SemaphoreType.DMA((_NBUF,)),
            pltpu.SemaphoreType.DMA((_NOUT,)),
        ],
    )(logits.reshape(B, 1, D), uniform)
    return out.reshape(B, D)
